# baseline (speedup 1.0000x reference)
import math

import jax
import jax.numpy as jnp
from jax.experimental import pallas as pl

_RATIO = 0.5


def _gcn_conv(x, src, dst, edge_weight, W, b, num_nodes):
    loop = jnp.arange(num_nodes, dtype=src.dtype)
    src_f = jnp.concatenate([src, loop])
    dst_f = jnp.concatenate([dst, loop])
    ew = jnp.concatenate([edge_weight, jnp.full((num_nodes,), 1.0, dtype=x.dtype)])
    deg = jnp.zeros((num_nodes,), x.dtype).at[dst_f].add(ew)
    dinv = jnp.where(deg > 0, 1.0 / jnp.sqrt(deg), 0.0)
    norm = dinv[src_f] * ew * dinv[dst_f]
    xw = x @ W
    out = jnp.zeros((num_nodes, W.shape[1]), x.dtype).at[dst_f].add(norm[:, None] * xw[src_f])
    return out + b


def _topk_pool(x, src, dst, edge_weight, p):
    n = x.shape[0]
    score = jnp.tanh((x @ p) / jnp.linalg.norm(p))
    k = int(math.ceil(_RATIO * n))
    topv, perm = jax.lax.top_k(score, k)
    x_new = x[perm] * topv[:, None]
    mapping = jnp.full((n,), -1, dtype=jnp.int32).at[perm].set(jnp.arange(k, dtype=jnp.int32))
    r = mapping[src]
    c = mapping[dst]
    valid = (r >= 0) & (c >= 0)
    src_new = jnp.where(valid, r, 0)
    dst_new = jnp.where(valid, c, 0)
    ew_new = edge_weight * valid.astype(x.dtype)
    return x_new, src_new, dst_new, ew_new, perm, k


def _unpool(xl, idxs, n_up):
    return jnp.zeros((n_up, xl.shape[1]), xl.dtype).at[idxs].set(xl)


def _touch_kernel(x_ref, o_ref):
    o_ref[...] = x_ref[...] * 1.0


def kernel(x, edge_index, batch, W1, b1, W2, b2, W3, b3, W4, b4, W5, b5, W6, b6, W7, b7, p1, p2, p3, l1_W, cls_W, cls_b):
    elu = jax.nn.elu
    src = edge_index[0]
    dst = edge_index[1]
    ew = jnp.ones((src.shape[0],), x.dtype)
    n0 = x.shape[0]
    x1 = elu(_gcn_conv(x, src, dst, ew, W1, b1, n0))
    x2, src2, dst2, ew2, idx2, k2 = _topk_pool(x1, src, dst, ew, p1)
    x2 = elu(x2)
    x3 = elu(_gcn_conv(x2, src2, dst2, ew2, W2, b2, k2))
    x4, src4, dst4, ew4, idx4, k4 = _topk_pool(x3, src2, dst2, ew2, p2)
    x4 = elu(x4)
    x5 = elu(_gcn_conv(x4, src4, dst4, ew4, W3, b3, k4))
    x6, src6, dst6, ew6, idx6, k6 = _topk_pool(x5, src4, dst4, ew4, p3)
    x6 = elu(x6)
    x7 = elu(_gcn_conv(x6, src6, dst6, ew6, W4, b4, k6))
    x8 = elu(jnp.concatenate([_unpool(x7, idx6, k4), x5], axis=1))
    x9 = elu(_gcn_conv(x8, src4, dst4, ew4, W5, b5, k4))
    x10 = elu(jnp.concatenate([_unpool(x9, idx4, k2), x3], axis=1))
    x11 = elu(_gcn_conv(x10, src2, dst2, ew2, W6, b6, k2))
    x12 = elu(jnp.concatenate([_unpool(x11, idx2, n0), x1], axis=1))
    x13 = elu(_gcn_conv(x12, src, dst, ew, W7, b7, n0))
    gmax = jax.ops.segment_max(x13, batch, num_segments=1)
    gsum = jax.ops.segment_sum(x13, batch, num_segments=1)
    cnt = jax.ops.segment_sum(jnp.ones((x13.shape[0], 1), x.dtype), batch, num_segments=1)
    g = elu(jnp.concatenate([gmax, gsum / cnt], axis=1))
    g = elu(g @ l1_W)
    logits = g @ cls_W + cls_b
    logits = pl.pallas_call(
        _touch_kernel,
        out_shape=jax.ShapeDtypeStruct(logits.shape, logits.dtype),
    )(logits)
    return jax.nn.log_softmax(logits, axis=-1)


# speedup vs baseline: 7.6643x; 7.6643x over previous
import functools
import math

import jax
import jax.numpy as jnp
import numpy as np
from jax import lax
from jax.experimental import pallas as pl
from jax.experimental.pallas import tpu as pltpu
from jax.experimental.pallas import tpu_sc as plsc

_N = 10000
_NP = 10240
_E = 320000
_NW = 32
_C = 128
_EP = 327680
_CHUNKS = _EP // _NW // _C
_RPW = _NP // 16
_ZR = 128


@functools.lru_cache(None)
def _make_agg(d):
    mesh = plsc.VectorSubcoreMesh(core_axis_name="c", subcore_axis_name="s")

    @functools.partial(
        pl.kernel,
        out_type=jax.ShapeDtypeStruct((2 * _NP, d), jnp.float32),
        mesh=mesh,
        scratch_types=[
            pltpu.VMEM((_C,), jnp.int32),
            pltpu.VMEM((_C,), jnp.int32),
            pltpu.VMEM((_C, d), jnp.float32),
            pltpu.VMEM((_ZR, d), jnp.float32),
            pltpu.VMEM_SHARED((_NP, d), jnp.float32),
            pltpu.SemaphoreType.DMA,
            pltpu.SemaphoreType.DMA,
        ],
    )
    def agg(xw_hbm, src_hbm, dst_hbm, out_hbm, srcb, dstb, rows, zbuf, acc,
            sem_i, sem_g):
        cid = lax.axis_index("c")
        sid = lax.axis_index("s")
        wid = sid * 2 + cid

        def zrow(r, carry):
            for j in range(d // 16):
                zbuf[r, pl.ds(j * 16, 16)] = jnp.zeros((16,), jnp.float32)
            return carry

        lax.fori_loop(0, _ZR, zrow, 0)
        base_r = sid * _RPW
        for t in range(_RPW // _ZR):
            pltpu.sync_copy(zbuf, acc.at[pl.ds(base_r + t * _ZR, _ZR)])
        plsc.subcore_barrier()

        ebase = wid * (_EP // _NW)

        def chunk(g, carry):
            off = ebase + g * _C
            cp_s = pltpu.async_copy(src_hbm.at[pl.ds(off, _C)], srcb, sem_i)
            cp_d = pltpu.async_copy(dst_hbm.at[pl.ds(off, _C)], dstb, sem_i)
            cp_s.wait()
            cp_d.wait()
            pltpu.async_copy(xw_hbm.at[srcb], rows, sem_g).wait()
            pltpu.sync_copy(rows, acc.at[dstb], add=True)
            return carry

        lax.fori_loop(0, _CHUNKS, chunk, 0)
        plsc.subcore_barrier()
        pltpu.sync_copy(acc.at[pl.ds(base_r, _RPW)],
                        out_hbm.at[pl.ds(cid * _NP + base_r, _RPW)])

    return agg


_F32 = jnp.float32


def _elu(v):
    return jnp.where(v > 0, v, jnp.exp(jnp.minimum(v, 0.0)) - 1.0)


def _dinv_of(sraw_ref, m_ref):
    s = sraw_ref[0:_N, 0:1] + sraw_ref[_NP:_NP + _N, 0:1]
    return jnp.where(m_ref[...] > 0, lax.rsqrt(1.0 + s), 0.0)


def _mm1_body(x_ref, w_ref, sraw_ref, m_ref, o_ref):
    dinv = _dinv_of(sraw_ref, m_ref)
    o_ref[...] = jnp.dot(x_ref[...], w_ref[...],
                         preferred_element_type=_F32) * dinv


def _mm2_body(u_ref, v_ref, mu_ref, wa_ref, wb_ref, sraw_ref, m_ref, o_ref):
    dinv = _dinv_of(sraw_ref, m_ref)
    xw = (jnp.dot(_elu(u_ref[...] * mu_ref[...]), wa_ref[...],
                  preferred_element_type=_F32)
          + jnp.dot(_elu(v_ref[...]), wb_ref[...],
                    preferred_element_type=_F32))
    o_ref[...] = xw * dinv


def _mm3_body(ua_ref, ub_ref, v_ref, mu_ref, wa_ref, wb_ref, wc_ref,
              sraw_ref, m_ref, o_ref):
    dinv = _dinv_of(sraw_ref, m_ref)
    xw = (jnp.dot(_elu(ua_ref[...] * mu_ref[...]), wa_ref[...],
                  preferred_element_type=_F32)
          + jnp.dot(_elu(ub_ref[...] * mu_ref[...]), wb_ref[...],
                    preferred_element_type=_F32)
          + jnp.dot(_elu(v_ref[...]), wc_ref[...],
                    preferred_element_type=_F32))
    o_ref[...] = xw * dinv


def _ep_body(acc_ref, xw_ref, sraw_ref, m_ref, b_ref, o_ref):
    d = o_ref.shape[1]
    dinv = _dinv_of(sraw_ref, m_ref)
    agg = acc_ref[0:_N, 0:d] + acc_ref[_NP:_NP + _N, 0:d]
    o_ref[...] = _elu(dinv * (agg + xw_ref[0:_N, 0:d]) + b_ref[...])


def _ps_body(h_ref, p_ref, m_ref, o_ref):
    p = p_ref[...]
    nrm = jnp.sqrt(jnp.sum(p * p))
    z = jnp.dot(h_ref[...], p, preferred_element_type=_F32) / nrm
    o_ref[...] = jnp.where(m_ref[...] > 0, jnp.tanh(z), -2.0)


def _okey(score):
    bits = lax.bitcast_convert_type(score, jnp.int32)
    return jnp.where(bits < 0, bits ^ np.int32(0x7FFFFFFF), bits)


def _tk_body(s_ref, t_ref, j_ref, *, k):
    okey = _okey(s_ref[...])
    lo0 = np.int32(-1073741826)
    hi0 = np.int32(1069547520)

    def it(i, lh):
        lo, hi = lh
        mid = lo + (hi - lo) // 2
        c = jnp.sum((okey > mid).astype(jnp.int32))
        big = c >= k
        return jnp.where(big, mid, lo), jnp.where(big, hi, mid)

    _, thr = lax.fori_loop(0, 32, it, (jnp.int32(lo0), jnp.int32(hi0)))
    cnt_gt = jnp.sum((okey > thr).astype(jnp.int32))
    need = k - cnt_gt
    tie = okey == thr
    idx = (lax.broadcasted_iota(jnp.int32, (80, 128), 0) * 128
           + lax.broadcasted_iota(jnp.int32, (80, 128), 1))

    def it2(i, lh):
        lo, hi = lh
        mid = (lo + hi) // 2
        c = jnp.sum((tie & (idx < mid)).astype(jnp.int32))
        good = c >= need
        return jnp.where(good, lo, mid + 1), jnp.where(good, mid, hi)

    _, jthr = lax.fori_loop(0, 14, it2, (jnp.int32(0), jnp.int32(_NP)))
    t_ref[...] = jnp.full((1, 1), thr, jnp.int32)
    j_ref[...] = jnp.full((1, 1), jthr, jnp.int32)


def _ap_body(h_ref, sc_ref, t_ref, j_ref, x_ref, m_ref, m16_ref):
    score = sc_ref[...]
    okey = _okey(score)
    thr = t_ref[0, 0]
    jthr = j_ref[0, 0]
    idx = lax.broadcasted_iota(jnp.int32, (_N, 1), 0)
    sel = (okey > thr) | ((okey == thr) & (idx < jthr))
    mf = sel.astype(_F32)
    m_ref[...] = mf
    m16_ref[...] = jnp.broadcast_to(mf, (_N, 128))
    x_ref[...] = _elu(h_ref[...] * score * mf)


def _fin_body(x_ref, l1_ref, cw_ref, cb_ref, o_ref):
    x = x_ref[...]
    gmax = jnp.max(x, axis=0, keepdims=True)
    gmean = jnp.sum(x, axis=0, keepdims=True) / _N
    g = _elu(jnp.concatenate([gmax, gmean], axis=1))
    g = _elu(jnp.dot(g, l1_ref[...], preferred_element_type=_F32))
    logits = jnp.dot(g, cw_ref[...], preferred_element_type=_F32) + cb_ref[...]
    o_ref[...] = jax.nn.log_softmax(logits, axis=-1)


def _tc(body, out_shapes, *args):
    return pl.pallas_call(body, out_shape=out_shapes)(*args)


def kernel(x, edge_index, batch, W1, b1, W2, b2, W3, b3, W4, b4, W5, b5,
           W6, b6, W7, b7, p1, p2, p3, l1_W, cls_W, cls_b):
    f = jax.ShapeDtypeStruct
    src = jnp.concatenate([edge_index[0],
                           jnp.zeros((_EP - _E,), jnp.int32)])
    dst = jnp.concatenate([edge_index[1],
                           jnp.full((_EP - _E,), _N + 100, jnp.int32)])

    m0 = jnp.ones((_N, 1), _F32)
    m16_0 = jnp.ones((_N, 128), _F32)

    def sc_agg(xw):
        return _make_agg(128)(xw, src, dst)

    def pad128(w):
        return jnp.pad(w, ((0, 0), (0, 128 - w.shape[1])))

    def conv_pre(body, ops, W_blocks, sraw, m, dout):
        return _tc(body, f((_N, 128), _F32),
                   *ops, *[pad128(w) for w in W_blocks], sraw, m)

    def conv_post(acc, xw, sraw, m, b):
        return _tc(_ep_body, f((_N, b.shape[0]), _F32),
                   acc, xw, sraw, m, b.reshape(1, -1))

    def pool(h, m, p, k):
        score = _tc(_ps_body, f((_N, 1), _F32), h, p.reshape(-1, 1), m)
        s128 = jnp.pad(score[:, 0], (0, _NP - _N),
                       constant_values=-2.0).reshape(80, 128)
        thr, jthr = _tc(functools.partial(_tk_body, k=k),
                        [f((1, 1), jnp.int32), f((1, 1), jnp.int32)], s128)
        return _tc(_ap_body,
                   [f((_N, h.shape[1]), _F32), f((_N, 1), _F32),
                    f((_N, 128), _F32)],
                   h, score, thr, jthr)

    s0 = sc_agg(m16_0)
    xw1 = conv_pre(_mm1_body, [x], [W1], s0, m0, 32)
    x1 = conv_post(sc_agg(xw1), xw1, s0, m0, b1)
    x2, m2, m16_2 = pool(x1, m0, p1, 5000)
    s2 = sc_agg(m16_2)
    xw2 = conv_pre(_mm1_body, [x2], [W2], s2, m2, 64)
    x3 = conv_post(sc_agg(xw2), xw2, s2, m2, b2)
    x4, m4, m16_4 = pool(x3, m2, p2, 2500)
    s4 = sc_agg(m16_4)
    xw3 = conv_pre(_mm1_body, [x4], [W3], s4, m4, 128)
    x5 = conv_post(sc_agg(xw3), xw3, s4, m4, b3)
    x6, m6, m16_6 = pool(x5, m4, p3, 1250)
    s6 = sc_agg(m16_6)
    xw4a = conv_pre(_mm1_body, [x6], [W4[:, :128]], s6, m6, 128)
    xw4b = conv_pre(_mm1_body, [x6], [W4[:, 128:]], s6, m6, 128)
    x7a = conv_post(sc_agg(xw4a), xw4a, s6, m6, b4[:128])
    x7b = conv_post(sc_agg(xw4b), xw4b, s6, m6, b4[128:])
    xw5 = _tc(_mm3_body, f((_N, 128), _F32),
              x7a, x7b, x5, m6,
              pad128(W5[:128]), pad128(W5[128:256]), pad128(W5[256:]),
              s4, m4)
    x9 = conv_post(sc_agg(xw5), xw5, s4, m4, b5)
    xw6 = _tc(_mm2_body, f((_N, 128), _F32),
              x9, x3, m4, pad128(W6[:128]), pad128(W6[128:]), s2, m2)
    x11 = conv_post(sc_agg(xw6), xw6, s2, m2, b6)
    xw7 = _tc(_mm2_body, f((_N, 128), _F32),
              x11, x1, m2, pad128(W7[:64]), pad128(W7[64:]), s0, m0)
    x13 = conv_post(sc_agg(xw7), xw7, s0, m0, b7)
    return _tc(_fin_body, f((1, 10), _F32),
               x13, l1_W, cls_W, cls_b.reshape(1, -1))


# speedup vs baseline: 8.6557x; 1.1293x over previous
import functools
import math

import jax
import jax.numpy as jnp
import numpy as np
from jax import lax
from jax.experimental import pallas as pl
from jax.experimental.pallas import tpu as pltpu
from jax.experimental.pallas import tpu_sc as plsc

_N = 10000
_NP = 10240
_E = 320000
_NW = 32
_C = 128
_EP = 327680
_CHUNKS = _EP // _NW // _C
_RPW = _NP // 16
_ZR = 16


@functools.lru_cache(None)
def _make_agg(d):
    mesh = plsc.VectorSubcoreMesh(core_axis_name="c", subcore_axis_name="s")

    @functools.partial(
        pl.kernel,
        out_type=jax.ShapeDtypeStruct((2 * _NP, d), jnp.float32),
        mesh=mesh,
        scratch_types=[
            pltpu.VMEM((2, _C), jnp.int32),
            pltpu.VMEM((4, _C), jnp.int32),
            pltpu.VMEM((2, _C, d), jnp.float32),
            pltpu.VMEM((_ZR, d), jnp.float32),
            pltpu.VMEM_SHARED((_NP, d), jnp.float32),
            pltpu.SemaphoreType.DMA((2,)),
            pltpu.SemaphoreType.DMA((2,)),
            pltpu.SemaphoreType.DMA,
        ],
    )
    def agg(xw_hbm, src_hbm, dst_hbm, out_hbm, srcb, dstb, rows, zbuf,
            acc, gsem, ssem, isem):
        cid = lax.axis_index("c")
        sid = lax.axis_index("s")
        wid = sid * 2 + cid

        cps_is = [None] * _CHUNKS
        cps_id = [None] * _CHUNKS
        cps_g = [None] * _CHUNKS
        cps_s = [None] * _CHUNKS

        def load_idx(g):
            cps_is[g] = pltpu.async_copy(src_hbm.at[wid, g], srcb.at[g % 2],
                                         isem)
            cps_id[g] = pltpu.async_copy(dst_hbm.at[wid, g], dstb.at[g % 4],
                                         isem)

        def issue_gather(g):
            cps_g[g] = pltpu.async_copy(xw_hbm.at[srcb.at[g % 2]],
                                        rows.at[g % 2], gsem.at[g % 2])

        def issue_scatter(m):
            cps_s[m] = pltpu.async_copy(rows.at[m % 2], acc.at[dstb.at[m % 4]],
                                        ssem.at[m % 2], add=True)

        load_idx(0)
        load_idx(1)

        def zrow(r, carry):
            for j in range(d // 16):
                zbuf[r, pl.ds(j * 16, 16)] = jnp.zeros((16,), jnp.float32)
            return carry

        lax.fori_loop(0, _ZR, zrow, 0)
        base_r = sid * _RPW
        for t in range(_RPW // _ZR):
            pltpu.sync_copy(zbuf, acc.at[pl.ds(base_r + t * _ZR, _ZR)])
        plsc.subcore_barrier()

        cps_is[0].wait()
        cps_id[0].wait()
        issue_gather(0)
        for m in range(_CHUNKS):
            cps_g[m].wait()
            issue_scatter(m)
            if m + 1 < _CHUNKS:
                if m >= 1:
                    cps_s[m - 1].wait()
                cps_is[m + 1].wait()
                cps_id[m + 1].wait()
                issue_gather(m + 1)
                if m + 2 < _CHUNKS:
                    load_idx(m + 2)
        cps_s[_CHUNKS - 2].wait()
        cps_s[_CHUNKS - 1].wait()

        plsc.subcore_barrier()
        pltpu.sync_copy(acc.at[pl.ds(base_r, _RPW)],
                        out_hbm.at[pl.ds(cid * _NP + base_r, _RPW)])

    return agg


_F32 = jnp.float32


def _elu(v):
    return jnp.where(v > 0, v, jnp.exp(jnp.minimum(v, 0.0)) - 1.0)


def _dinv_of(sraw_ref, m_ref):
    s = sraw_ref[0:_N, 0:1] + sraw_ref[_NP:_NP + _N, 0:1]
    return jnp.where(m_ref[...] > 0, lax.rsqrt(1.0 + s), 0.0)


def _mm1_body(x_ref, w_ref, sraw_ref, m_ref, o_ref):
    dinv = _dinv_of(sraw_ref, m_ref)
    o_ref[...] = jnp.dot(x_ref[...], w_ref[...],
                         preferred_element_type=_F32) * dinv


def _mm2_body(u_ref, v_ref, mu_ref, wa_ref, wb_ref, sraw_ref, m_ref, o_ref):
    dinv = _dinv_of(sraw_ref, m_ref)
    xw = (jnp.dot(_elu(u_ref[...] * mu_ref[...]), wa_ref[...],
                  preferred_element_type=_F32)
          + jnp.dot(_elu(v_ref[...]), wb_ref[...],
                    preferred_element_type=_F32))
    o_ref[...] = xw * dinv


def _mm3_body(ua_ref, ub_ref, v_ref, mu_ref, wa_ref, wb_ref, wc_ref,
              sraw_ref, m_ref, o_ref):
    dinv = _dinv_of(sraw_ref, m_ref)
    xw = (jnp.dot(_elu(ua_ref[...] * mu_ref[...]), wa_ref[...],
                  preferred_element_type=_F32)
          + jnp.dot(_elu(ub_ref[...] * mu_ref[...]), wb_ref[...],
                    preferred_element_type=_F32)
          + jnp.dot(_elu(v_ref[...]), wc_ref[...],
                    preferred_element_type=_F32))
    o_ref[...] = xw * dinv


def _ep_body(acc_ref, xw_ref, sraw_ref, m_ref, b_ref, o_ref):
    d = o_ref.shape[1]
    dinv = _dinv_of(sraw_ref, m_ref)
    agg = acc_ref[0:_N, 0:d] + acc_ref[_NP:_NP + _N, 0:d]
    o_ref[...] = _elu(dinv * (agg + xw_ref[0:_N, 0:d]) + b_ref[...])


def _ps_body(h_ref, p_ref, m_ref, o_ref):
    p = p_ref[...]
    nrm = jnp.sqrt(jnp.sum(p * p))
    z = jnp.dot(h_ref[...], p, preferred_element_type=_F32) / nrm
    o_ref[...] = jnp.where(m_ref[...] > 0, jnp.tanh(z), -2.0)


def _okey(score):
    bits = lax.bitcast_convert_type(score, jnp.int32)
    return jnp.where(bits < 0, bits ^ np.int32(0x7FFFFFFF), bits)


def _tk_body(s_ref, t_ref, j_ref, *, k):
    okey = _okey(s_ref[...])
    lo0 = np.int32(-1073741826)
    hi0 = np.int32(1069547520)

    def it(i, lh):
        lo, hi = lh
        mid = lo + (hi - lo) // 2
        c = jnp.sum((okey > mid).astype(jnp.int32))
        big = c >= k
        return jnp.where(big, mid, lo), jnp.where(big, hi, mid)

    _, thr = lax.fori_loop(0, 32, it, (jnp.int32(lo0), jnp.int32(hi0)))
    cnt_gt = jnp.sum((okey > thr).astype(jnp.int32))
    need = k - cnt_gt
    tie = okey == thr
    idx = (lax.broadcasted_iota(jnp.int32, (80, 128), 0) * 128
           + lax.broadcasted_iota(jnp.int32, (80, 128), 1))

    def it2(i, lh):
        lo, hi = lh
        mid = (lo + hi) // 2
        c = jnp.sum((tie & (idx < mid)).astype(jnp.int32))
        good = c >= need
        return jnp.where(good, lo, mid + 1), jnp.where(good, mid, hi)

    _, jthr = lax.fori_loop(0, 14, it2, (jnp.int32(0), jnp.int32(_NP)))
    t_ref[...] = jnp.full((1, 1), thr, jnp.int32)
    j_ref[...] = jnp.full((1, 1), jthr, jnp.int32)


def _ap_body(h_ref, sc_ref, t_ref, j_ref, x_ref, m_ref, m16_ref):
    score = sc_ref[...]
    okey = _okey(score)
    thr = t_ref[0, 0]
    jthr = j_ref[0, 0]
    idx = lax.broadcasted_iota(jnp.int32, (_N, 1), 0)
    sel = (okey > thr) | ((okey == thr) & (idx < jthr))
    mf = sel.astype(_F32)
    m_ref[...] = mf
    m16_ref[...] = jnp.broadcast_to(mf, (_N, 128))
    x_ref[...] = _elu(h_ref[...] * score * mf)


def _fin_body(x_ref, l1_ref, cw_ref, cb_ref, o_ref):
    x = x_ref[...]
    gmax = jnp.max(x, axis=0, keepdims=True)
    gmean = jnp.sum(x, axis=0, keepdims=True) / _N
    g = _elu(jnp.concatenate([gmax, gmean], axis=1))
    g = _elu(jnp.dot(g, l1_ref[...], preferred_element_type=_F32))
    logits = jnp.dot(g, cw_ref[...], preferred_element_type=_F32) + cb_ref[...]
    o_ref[...] = jax.nn.log_softmax(logits, axis=-1)


def _tc(body, out_shapes, *args):
    return pl.pallas_call(body, out_shape=out_shapes)(*args)


def kernel(x, edge_index, batch, W1, b1, W2, b2, W3, b3, W4, b4, W5, b5,
           W6, b6, W7, b7, p1, p2, p3, l1_W, cls_W, cls_b):
    f = jax.ShapeDtypeStruct
    src = jnp.concatenate([edge_index[0],
                           jnp.zeros((_EP - _E,), jnp.int32)])
    dst = jnp.concatenate([edge_index[1],
                           jnp.full((_EP - _E,), _N + 100, jnp.int32)])
    src = src.reshape(_NW, _CHUNKS, _C)
    dst = dst.reshape(_NW, _CHUNKS, _C)

    m0 = jnp.ones((_N, 1), _F32)
    m16_0 = jnp.ones((_N, 128), _F32)

    def sc_agg(xw):
        return _make_agg(128)(xw, src, dst)

    def pad128(w):
        return jnp.pad(w, ((0, 0), (0, 128 - w.shape[1])))

    def conv_pre(body, ops, W_blocks, sraw, m, dout):
        return _tc(body, f((_N, 128), _F32),
                   *ops, *[pad128(w) for w in W_blocks], sraw, m)

    def conv_post(acc, xw, sraw, m, b):
        return _tc(_ep_body, f((_N, b.shape[0]), _F32),
                   acc, xw, sraw, m, b.reshape(1, -1))

    def pool(h, m, p, k):
        score = _tc(_ps_body, f((_N, 1), _F32), h, p.reshape(-1, 1), m)
        s128 = jnp.pad(score[:, 0], (0, _NP - _N),
                       constant_values=-2.0).reshape(80, 128)
        thr, jthr = _tc(functools.partial(_tk_body, k=k),
                        [f((1, 1), jnp.int32), f((1, 1), jnp.int32)], s128)
        return _tc(_ap_body,
                   [f((_N, h.shape[1]), _F32), f((_N, 1), _F32),
                    f((_N, 128), _F32)],
                   h, score, thr, jthr)

    s0 = sc_agg(m16_0)
    xw1 = conv_pre(_mm1_body, [x], [W1], s0, m0, 32)
    x1 = conv_post(sc_agg(xw1), xw1, s0, m0, b1)
    x2, m2, m16_2 = pool(x1, m0, p1, 5000)
    s2 = sc_agg(m16_2)
    xw2 = conv_pre(_mm1_body, [x2], [W2], s2, m2, 64)
    x3 = conv_post(sc_agg(xw2), xw2, s2, m2, b2)
    x4, m4, m16_4 = pool(x3, m2, p2, 2500)
    s4 = sc_agg(m16_4)
    xw3 = conv_pre(_mm1_body, [x4], [W3], s4, m4, 128)
    x5 = conv_post(sc_agg(xw3), xw3, s4, m4, b3)
    x6, m6, m16_6 = pool(x5, m4, p3, 1250)
    s6 = sc_agg(m16_6)
    xw4a = conv_pre(_mm1_body, [x6], [W4[:, :128]], s6, m6, 128)
    xw4b = conv_pre(_mm1_body, [x6], [W4[:, 128:]], s6, m6, 128)
    x7a = conv_post(sc_agg(xw4a), xw4a, s6, m6, b4[:128])
    x7b = conv_post(sc_agg(xw4b), xw4b, s6, m6, b4[128:])
    xw5 = _tc(_mm3_body, f((_N, 128), _F32),
              x7a, x7b, x5, m6,
              pad128(W5[:128]), pad128(W5[128:256]), pad128(W5[256:]),
              s4, m4)
    x9 = conv_post(sc_agg(xw5), xw5, s4, m4, b5)
    xw6 = _tc(_mm2_body, f((_N, 128), _F32),
              x9, x3, m4, pad128(W6[:128]), pad128(W6[128:]), s2, m2)
    x11 = conv_post(sc_agg(xw6), xw6, s2, m2, b6)
    xw7 = _tc(_mm2_body, f((_N, 128), _F32),
              x11, x1, m2, pad128(W7[:64]), pad128(W7[64:]), s0, m0)
    x13 = conv_post(sc_agg(xw7), xw7, s0, m0, b7)
    return _tc(_fin_body, f((1, 10), _F32),
               x13, l1_W, cls_W, cls_b.reshape(1, -1))


# speedup vs baseline: 18.0148x; 2.0813x over previous
import functools
import math

import jax
import jax.numpy as jnp
import numpy as np
from jax import lax
from jax.experimental import pallas as pl
from jax.experimental.pallas import tpu as pltpu
from jax.experimental.pallas import tpu_sc as plsc

_N = 10000
_NP = 10240
_E = 320000
_NW = 32
_C = 128
_EP = 327680
_CHUNKS = _EP // _NW // _C
_RPW = _NP // 16
_ZR = 16


_NIDX = 16


@functools.lru_cache(None)
def _make_agg(d):
    mesh = plsc.VectorSubcoreMesh(core_axis_name="c", subcore_axis_name="s")
    budget = (2097151 - _NP * d) // 16 - 6144
    nb = max(2, min(8, budget // (_C * d)))
    nb = nb if nb % 2 == 0 else nb - 1
    ahead = nb // 2
    ip = ahead + 2

    @functools.partial(
        pl.kernel,
        out_type=jax.ShapeDtypeStruct((2 * _NP, d), jnp.float32),
        mesh=mesh,
        compiler_params=pltpu.CompilerParams(use_tc_tiling_on_sc=False),
        scratch_types=[
            pltpu.VMEM((_NIDX, _C), jnp.int32),
            pltpu.VMEM((_NIDX, _C), jnp.int32),
            pltpu.VMEM((nb, _C, d), jnp.float32),
            pltpu.VMEM((_ZR, d), jnp.float32),
            pltpu.VMEM_SHARED((_NP, d), jnp.float32),
            pltpu.SemaphoreType.DMA((nb,)),
            pltpu.SemaphoreType.DMA((nb,)),
            pltpu.SemaphoreType.DMA,
        ],
    )
    def agg(xw_hbm, src_hbm, dst_hbm, out_hbm, srcb, dstb, rows, zbuf,
            acc, gsem, ssem, isem):
        cid = lax.axis_index("c")
        sid = lax.axis_index("s")
        wid = sid * 2 + cid

        cps_is = [None] * _CHUNKS
        cps_id = [None] * _CHUNKS
        cps_g = [None] * _CHUNKS
        cps_s = [None] * _CHUNKS

        def load_idx(g):
            cps_is[g] = pltpu.async_copy(src_hbm.at[wid, g],
                                         srcb.at[g % _NIDX], isem)
            cps_id[g] = pltpu.async_copy(dst_hbm.at[wid, g],
                                         dstb.at[g % _NIDX], isem)

        def issue_gather(g):
            cps_g[g] = pltpu.async_copy(xw_hbm.at[srcb.at[g % _NIDX]],
                                        rows.at[g % nb], gsem.at[g % nb])

        def issue_scatter(m):
            cps_s[m] = pltpu.async_copy(rows.at[m % nb],
                                        acc.at[dstb.at[m % _NIDX]],
                                        ssem.at[m % nb], add=True)

        for g in range(min(ip, _CHUNKS)):
            load_idx(g)

        def zrow(r, carry):
            for j in range(d // 16):
                zbuf[r, pl.ds(j * 16, 16)] = jnp.zeros((16,), jnp.float32)
            return carry

        lax.fori_loop(0, _ZR, zrow, 0)
        base_r = sid * _RPW
        for t in range(_RPW // _ZR):
            pltpu.sync_copy(zbuf, acc.at[pl.ds(base_r + t * _ZR, _ZR)])
        plsc.subcore_barrier()

        for g in range(min(ahead, _CHUNKS)):
            cps_is[g].wait()
            issue_gather(g)
        for m in range(_CHUNKS):
            cps_g[m].wait()
            cps_id[m].wait()
            issue_scatter(m)
            g = m + ahead
            if g < _CHUNKS:
                if m >= ahead:
                    cps_s[m - ahead].wait()
                cps_is[g].wait()
                issue_gather(g)
                if g + 2 < _CHUNKS:
                    load_idx(g + 2)
        for t in range(max(0, _CHUNKS - 2 * ahead), _CHUNKS):
            cps_s[t].wait()

        plsc.subcore_barrier()
        pltpu.sync_copy(acc.at[pl.ds(base_r, _RPW)],
                        out_hbm.at[pl.ds(cid * _NP + base_r, _RPW)])

    return agg


_F32 = jnp.float32


def _elu(v):
    return jnp.where(v > 0, v, jnp.exp(jnp.minimum(v, 0.0)) - 1.0)


def _dinv_of(sraw_ref, m_ref):
    s = sraw_ref[0:_N, 0:1] + sraw_ref[_NP:_NP + _N, 0:1]
    return jnp.where(m_ref[...] > 0, lax.rsqrt(1.0 + s), 0.0)


def _mm1_body(x_ref, w_ref, sraw_ref, m_ref, o_ref):
    dinv = _dinv_of(sraw_ref, m_ref)
    o_ref[...] = jnp.dot(x_ref[...], w_ref[...],
                         preferred_element_type=_F32) * dinv


def _mm2_body(u_ref, v_ref, mu_ref, wa_ref, wb_ref, sraw_ref, m_ref, o_ref):
    dinv = _dinv_of(sraw_ref, m_ref)
    xw = (jnp.dot(_elu(u_ref[...] * mu_ref[...]), wa_ref[...],
                  preferred_element_type=_F32)
          + jnp.dot(_elu(v_ref[...]), wb_ref[...],
                    preferred_element_type=_F32))
    o_ref[...] = xw * dinv


def _mm3_body(ua_ref, ub_ref, v_ref, mu_ref, wa_ref, wb_ref, wc_ref,
              sraw_ref, m_ref, o_ref):
    dinv = _dinv_of(sraw_ref, m_ref)
    xw = (jnp.dot(_elu(ua_ref[...] * mu_ref[...]), wa_ref[...],
                  preferred_element_type=_F32)
          + jnp.dot(_elu(ub_ref[...] * mu_ref[...]), wb_ref[...],
                    preferred_element_type=_F32)
          + jnp.dot(_elu(v_ref[...]), wc_ref[...],
                    preferred_element_type=_F32))
    o_ref[...] = xw * dinv


def _ep_body(acc_ref, xw_ref, sraw_ref, m_ref, b_ref, o_ref):
    d = o_ref.shape[1]
    dinv = _dinv_of(sraw_ref, m_ref)
    agg = acc_ref[0:_N, 0:d] + acc_ref[_NP:_NP + _N, 0:d]
    o_ref[...] = _elu(dinv * (agg + xw_ref[0:_N, 0:d]) + b_ref[...])


def _ps_body(h_ref, p_ref, m_ref, o_ref):
    p = p_ref[...]
    nrm = jnp.sqrt(jnp.sum(p * p))
    z = jnp.dot(h_ref[...], p, preferred_element_type=_F32) / nrm
    o_ref[...] = jnp.where(m_ref[...] > 0, jnp.tanh(z), -2.0)


def _okey(score):
    bits = lax.bitcast_convert_type(score, jnp.int32)
    return jnp.where(bits < 0, bits ^ np.int32(0x7FFFFFFF), bits)


def _tk_body(s_ref, t_ref, j_ref, *, k):
    okey = _okey(s_ref[...])
    lo0 = np.int32(-1073741826)
    hi0 = np.int32(1069547520)

    def it(i, lh):
        lo, hi = lh
        mid = lo + (hi - lo) // 2
        c = jnp.sum((okey > mid).astype(jnp.int32))
        big = c >= k
        return jnp.where(big, mid, lo), jnp.where(big, hi, mid)

    _, thr = lax.fori_loop(0, 32, it, (jnp.int32(lo0), jnp.int32(hi0)))
    cnt_gt = jnp.sum((okey > thr).astype(jnp.int32))
    need = k - cnt_gt
    tie = okey == thr
    idx = (lax.broadcasted_iota(jnp.int32, (80, 128), 0) * 128
           + lax.broadcasted_iota(jnp.int32, (80, 128), 1))

    def it2(i, lh):
        lo, hi = lh
        mid = (lo + hi) // 2
        c = jnp.sum((tie & (idx < mid)).astype(jnp.int32))
        good = c >= need
        return jnp.where(good, lo, mid + 1), jnp.where(good, mid, hi)

    _, jthr = lax.fori_loop(0, 14, it2, (jnp.int32(0), jnp.int32(_NP)))
    t_ref[...] = jnp.full((1, 1), thr, jnp.int32)
    j_ref[...] = jnp.full((1, 1), jthr, jnp.int32)


def _ap_body(h_ref, sc_ref, t_ref, j_ref, x_ref, m_ref, m16_ref):
    score = sc_ref[...]
    okey = _okey(score)
    thr = t_ref[0, 0]
    jthr = j_ref[0, 0]
    idx = lax.broadcasted_iota(jnp.int32, (_N, 1), 0)
    sel = (okey > thr) | ((okey == thr) & (idx < jthr))
    mf = sel.astype(_F32)
    m_ref[...] = mf
    m16_ref[...] = jnp.broadcast_to(mf, (_N, 16))
    x_ref[...] = _elu(h_ref[...] * score * mf)


def _fin_body(x_ref, l1_ref, cw_ref, cb_ref, o_ref):
    x = x_ref[...]
    gmax = jnp.max(x, axis=0, keepdims=True)
    gmean = jnp.sum(x, axis=0, keepdims=True) / _N
    g = _elu(jnp.concatenate([gmax, gmean], axis=1))
    g = _elu(jnp.dot(g, l1_ref[...], preferred_element_type=_F32))
    logits = jnp.dot(g, cw_ref[...], preferred_element_type=_F32) + cb_ref[...]
    o_ref[...] = jax.nn.log_softmax(logits, axis=-1)


def _tc(body, out_shapes, *args):
    return pl.pallas_call(body, out_shape=out_shapes)(*args)


def kernel(x, edge_index, batch, W1, b1, W2, b2, W3, b3, W4, b4, W5, b5,
           W6, b6, W7, b7, p1, p2, p3, l1_W, cls_W, cls_b):
    f = jax.ShapeDtypeStruct
    src = jnp.concatenate([edge_index[0],
                           jnp.zeros((_EP - _E,), jnp.int32)])
    dst = jnp.concatenate([edge_index[1],
                           jnp.full((_EP - _E,), _N + 100, jnp.int32)])
    src = src.reshape(_NW, _CHUNKS, _C)
    dst = dst.reshape(_NW, _CHUNKS, _C)

    m0 = jnp.ones((_N, 1), _F32)
    m16_0 = jnp.ones((_N, 16), _F32)

    def sc_agg(xw):
        return _make_agg(xw.shape[1])(xw, src, dst)

    def conv_pre(body, ops, W_blocks, sraw, m, dout):
        return _tc(body, f((_N, dout), _F32), *ops, *W_blocks, sraw, m)

    def conv_post(acc, xw, sraw, m, b):
        return _tc(_ep_body, f((_N, b.shape[0]), _F32),
                   acc, xw, sraw, m, b.reshape(1, -1))

    def pool(h, m, p, k):
        score = _tc(_ps_body, f((_N, 1), _F32), h, p.reshape(-1, 1), m)
        s128 = jnp.pad(score[:, 0], (0, _NP - _N),
                       constant_values=-2.0).reshape(80, 128)
        thr, jthr = _tc(functools.partial(_tk_body, k=k),
                        [f((1, 1), jnp.int32), f((1, 1), jnp.int32)], s128)
        return _tc(_ap_body,
                   [f((_N, h.shape[1]), _F32), f((_N, 1), _F32),
                    f((_N, 16), _F32)],
                   h, score, thr, jthr)

    s0 = sc_agg(m16_0)
    xw1 = conv_pre(_mm1_body, [x], [W1], s0, m0, 32)
    x1 = conv_post(sc_agg(xw1), xw1, s0, m0, b1)
    x2, m2, m16_2 = pool(x1, m0, p1, 5000)
    s2 = sc_agg(m16_2)
    xw2 = conv_pre(_mm1_body, [x2], [W2], s2, m2, 64)
    x3 = conv_post(sc_agg(xw2), xw2, s2, m2, b2)
    x4, m4, m16_4 = pool(x3, m2, p2, 2500)
    s4 = sc_agg(m16_4)
    xw3 = conv_pre(_mm1_body, [x4], [W3], s4, m4, 128)
    x5 = conv_post(sc_agg(xw3), xw3, s4, m4, b3)
    x6, m6, m16_6 = pool(x5, m4, p3, 1250)
    s6 = sc_agg(m16_6)
    xw4a = conv_pre(_mm1_body, [x6], [W4[:, :128]], s6, m6, 128)
    xw4b = conv_pre(_mm1_body, [x6], [W4[:, 128:]], s6, m6, 128)
    x7a = conv_post(sc_agg(xw4a), xw4a, s6, m6, b4[:128])
    x7b = conv_post(sc_agg(xw4b), xw4b, s6, m6, b4[128:])
    xw5 = _tc(_mm3_body, f((_N, 128), _F32),
              x7a, x7b, x5, m6, W5[:128], W5[128:256], W5[256:], s4, m4)
    x9 = conv_post(sc_agg(xw5), xw5, s4, m4, b5)
    xw6 = _tc(_mm2_body, f((_N, 64), _F32),
              x9, x3, m4, W6[:128], W6[128:], s2, m2)
    x11 = conv_post(sc_agg(xw6), xw6, s2, m2, b6)
    xw7 = _tc(_mm2_body, f((_N, 32), _F32),
              x11, x1, m2, W7[:64], W7[64:], s0, m0)
    x13 = conv_post(sc_agg(xw7), xw7, s0, m0, b7)
    return _tc(_fin_body, f((1, 10), _F32),
               x13, l1_W, cls_W, cls_b.reshape(1, -1))
